# baseline (speedup 1.0000x reference)
import functools
import jax
import jax.numpy as jnp
from jax.experimental import pallas as pl

N_NODES_C = 10000
N_EDGES_C = 320000
N_GRAPHS_C = 128
D_C = 128
EDGE_BLK = 640


def _node_matmul_body(h_ref, w_ref, asrc_ref, adst_ref, hs_ref, ssrc_ref, sdst_ref):
    hs = jnp.dot(h_ref[...], w_ref[...], preferred_element_type=jnp.float32)
    hs_ref[...] = hs
    ssrc_ref[...] = jnp.dot(hs, asrc_ref[...], preferred_element_type=jnp.float32)
    sdst_ref[...] = jnp.dot(hs, adst_ref[...], preferred_element_type=jnp.float32)


def _node_matmul(h, W, a_src, a_dst):
    n = h.shape[0]
    return pl.pallas_call(
        _node_matmul_body,
        out_shape=(
            jax.ShapeDtypeStruct((n, D_C), jnp.float32),
            jax.ShapeDtypeStruct((n, 1), jnp.float32),
            jax.ShapeDtypeStruct((n, 1), jnp.float32),
        ),
    )(h, W, a_src.reshape(D_C, 1), a_dst.reshape(D_C, 1))


def _edge_matmul_body(ea_ref, we_ref, ae_ref, e_ref, se_ref):
    e = jnp.dot(ea_ref[...], we_ref[...], preferred_element_type=jnp.float32)
    e_ref[...] = e
    se_ref[...] = jnp.dot(e, ae_ref[...], preferred_element_type=jnp.float32)


def _edge_matmul(edge_attr, We, a_e):
    m, ed = edge_attr.shape
    grid = m // EDGE_BLK
    return pl.pallas_call(
        _edge_matmul_body,
        grid=(grid,),
        in_specs=[
            pl.BlockSpec((EDGE_BLK, ed), lambda i: (i, 0)),
            pl.BlockSpec((ed, D_C), lambda i: (0, 0)),
            pl.BlockSpec((D_C, 1), lambda i: (0, 0)),
        ],
        out_specs=(
            pl.BlockSpec((EDGE_BLK, D_C), lambda i: (i, 0)),
            pl.BlockSpec((EDGE_BLK, 1), lambda i: (i, 0)),
        ),
        out_shape=(
            jax.ShapeDtypeStruct((m, D_C), jnp.float32),
            jax.ShapeDtypeStruct((m, 1), jnp.float32),
        ),
    )(edge_attr, We, a_e.reshape(D_C, 1))


def _final_body(h_ref, batch_ref, mf_ref, w1_ref, b1_ref, w2_ref, b2_ref,
                wq_ref, wk_ref, wv_ref,
                rw0, rb0, rw1, rb1, rw2, rb2, rw3, rb3, rw4, rb4,
                out_ref):
    h = h_ref[...]
    batch = batch_ref[...]
    gids = jax.lax.broadcasted_iota(jnp.int32, (1, N_GRAPHS_C), 1)
    onehot = (batch == gids).astype(jnp.float32)

    me = jnp.maximum(
        jnp.dot(mf_ref[...], w1_ref[...], preferred_element_type=jnp.float32)
        + b1_ref[...], 0.0)
    me = jnp.dot(me, w2_ref[...], preferred_element_type=jnp.float32) + b2_ref[...]

    q = jnp.dot(me, wq_ref[...], preferred_element_type=jnp.float32)
    k = jnp.dot(h, wk_ref[...], preferred_element_type=jnp.float32)
    v = jnp.dot(h, wv_ref[...], preferred_element_type=jnp.float32)

    qg = jnp.dot(onehot, q, preferred_element_type=jnp.float32)
    scores = jnp.sum(qg * k, axis=1, keepdims=True) * (1.0 / jnp.sqrt(128.0))
    smask = jnp.where(onehot > 0.0, scores, -jnp.inf)
    m = jnp.max(smask, axis=0, keepdims=True)
    m = jnp.where(jnp.isfinite(m), m, 0.0)
    mg = jnp.dot(onehot, m.T, preferred_element_type=jnp.float32)
    ex = jnp.exp(scores - mg)
    denom = jax.lax.dot_general(onehot, ex, (((0,), (0,)), ((), ())),
                                preferred_element_type=jnp.float32)
    dg = jnp.dot(onehot, denom, preferred_element_type=jnp.float32)
    alpha = ex / (dg + 1e-16)
    attn = jax.lax.dot_general(onehot, v * alpha, (((0,), (0,)), ((), ())),
                               preferred_element_type=jnp.float32)

    hc = jnp.concatenate([attn, me], axis=1)
    rws = [rw0, rw1, rw2, rw3, rw4]
    rbs = [rb0, rb1, rb2, rb3, rb4]
    for i in range(5):
        hc = jnp.dot(hc, rws[i][...], preferred_element_type=jnp.float32) + rbs[i][...]
        if i < 4:
            hc = jnp.maximum(hc, 0.0)
    out_ref[...] = hc


def _final_stage(h, batch, metal_features, params):
    mf = params['metal_fc']
    at = params['attn']
    reg = params['reg']
    args = [h, batch.astype(jnp.int32).reshape(-1, 1), metal_features,
            mf['W1'], mf['b1'].reshape(1, -1), mf['W2'], mf['b2'].reshape(1, -1),
            at['Wq'], at['Wk'], at['Wv']]
    for lp in reg:
        args.append(lp['W'])
        args.append(lp['b'].reshape(1, -1))
    return pl.pallas_call(
        _final_body,
        out_shape=jax.ShapeDtypeStruct((N_GRAPHS_C, 1), jnp.float32),
    )(*args)


def _edge_phase(hs, e, s_src, s_dst, se, src, dst, b, n_nodes):
    logits = s_src[src, 0] + s_dst[dst, 0] + se[:, 0]
    logits = jax.nn.leaky_relu(logits, 0.2)
    m = jax.ops.segment_max(logits, dst, num_segments=n_nodes)
    m = jnp.where(jnp.isfinite(m), m, 0.0)
    ex = jnp.exp(logits - m[dst])
    denom = jax.ops.segment_sum(ex, dst, num_segments=n_nodes)
    msg = (hs[src] + e) * ex[:, None]
    acc = jax.ops.segment_sum(msg, dst, num_segments=n_nodes)
    out = acc / (denom[:, None] + 1e-16) + b
    return jnp.where(out > 0.0, out, jnp.expm1(jnp.minimum(out, 0.0)))


@jax.jit
def _run(x, edge_attr, metal_features, params, edge_index, batch):
    src = edge_index[0].astype(jnp.int32)
    dst = edge_index[1].astype(jnp.int32)
    n_nodes = x.shape[0]
    h = x
    for p in params['paw']:
        hs, s_src, s_dst = _node_matmul(h, p['W'], p['a_src'], p['a_dst'])
        e, se = _edge_matmul(edge_attr, p['We'], p['a_e'])
        h = _edge_phase(hs, e, s_src, s_dst, se, src, dst, p['b'], n_nodes)
    out = _final_stage(h, batch, metal_features, params)
    return out[:, 0]


def kernel(x, edge_attr, metal_features, params, edge_index, batch):
    return _run(x, edge_attr, metal_features, params, edge_index, batch)
